# baseline (speedup 1.0000x reference)
import jax
import jax.numpy as jnp
from jax import lax
from jax.experimental import pallas as pl
from jax.experimental.pallas import tpu as pltpu
from jax.experimental.pallas import tpu_sc as plsc

_NUM_PRE = 2048
_NUM_POST = 1024
_BATCH = 16
_N = _NUM_PRE * _NUM_POST
_K_TARGET = int(0.1 * _N)
_DT = 0.1
_BLK = 256
_G = _NUM_PRE // _BLK

_SUB_ROWS = 64
_K_SUB = _K_TARGET // 32
_M_SUB = 480
_SUB_ITERS = 22
_RANK_TOL = 3
_INF_BITS = 0x7F800000

_NW = 32
_PER_W = _N // _NW
_CAP = 4096
_IMIN = jnp.iinfo(jnp.int32).min


def _sub_bisect(data, target, n_iter):
    def step(_, lohi):
        lo, hi = lohi
        mid = lo + (hi - lo) // 2
        c = jnp.sum((data >= mid).astype(jnp.int32))
        ge = c >= target
        return jnp.where(ge, mid, lo), jnp.where(ge, hi, mid)

    lo, _ = lax.fori_loop(0, n_iter, step, (jnp.int32(0),
                                            jnp.int32(_INF_BITS)))
    return lo


def _count_bisect(count_fn, lo0, hi0, c0):
    def cond(state):
        lo, hi, c_lo = state
        return jnp.logical_and(c_lo > _K_TARGET + _RANK_TOL, hi - lo > 1)

    def body(state):
        lo, hi, c_lo = state
        mid = lo + (hi - lo) // 2
        c = count_fn(mid)
        ge = c >= _K_TARGET
        return (jnp.where(ge, mid, lo), jnp.where(ge, hi, mid),
                jnp.where(ge, c, c_lo))

    lo, _, _ = lax.while_loop(cond, body, (lo0, hi0, c0))
    return lo


def _tc1_body(scal_ref, pre_blk_ref, post_ref, elig_blk_ref,
              ne_ref, abits_ref, meta_ref, smem):
    i = pl.program_id(0)
    decay = scal_ref[0]
    scale = scal_ref[1]
    thr = scal_ref[2]

    upd = jnp.dot(pre_blk_ref[...], post_ref[...],
                  preferred_element_type=jnp.float32) * scale
    ne = elig_blk_ref[...] * decay + upd
    ne_ref[...] = ne
    abits = jax.lax.bitcast_convert_type(jnp.abs(ne), jnp.int32)
    abits_ref[...] = abits
    cnt = jnp.sum((jnp.abs(ne) > thr).astype(jnp.int32))

    @pl.when(i == 0)
    def _first():
        sub = abits[0:_SUB_ROWS, :]
        t_lo = _sub_bisect(sub, _K_SUB + _M_SUB, _SUB_ITERS)
        t_hi = _sub_bisect(sub, _K_SUB - _M_SUB, _SUB_ITERS)
        smem[0] = cnt
        smem[1] = jnp.sum((abits >= t_hi).astype(jnp.int32))
        smem[2] = jnp.sum(jnp.logical_and(abits >= t_lo,
                                          abits < t_hi).astype(jnp.int32))
        smem[3] = t_lo
        smem[4] = t_hi

    @pl.when(i > 0)
    def _rest():
        t_lo = smem[3]
        t_hi = smem[4]
        smem[0] = smem[0] + cnt
        smem[1] = smem[1] + jnp.sum((abits >= t_hi).astype(jnp.int32))
        smem[2] = smem[2] + jnp.sum(
            jnp.logical_and(abits >= t_lo, abits < t_hi).astype(jnp.int32))

    @pl.when(i == _G - 1)
    def _last():
        ci = jax.lax.broadcasted_iota(jnp.int32, (1, 128), 1)
        meta_ref[...] = jnp.where(
            ci == 0, smem[3],
            jnp.where(ci == 1, smem[4],
                      jnp.where(ci == 2, smem[0],
                                jnp.where(ci == 3, smem[1],
                                          jnp.where(ci == 4, smem[2], 0)))))


def _tc1(scalars, pre_t, post, elig):
    out_shape = (
        jax.ShapeDtypeStruct((_NUM_PRE, _NUM_POST), jnp.float32),
        jax.ShapeDtypeStruct((_NUM_PRE, _NUM_POST), jnp.int32),
        jax.ShapeDtypeStruct((1, 128), jnp.int32),
    )
    return pl.pallas_call(
        _tc1_body,
        grid=(_G,),
        out_shape=out_shape,
        in_specs=[
            pl.BlockSpec(memory_space=pltpu.SMEM),
            pl.BlockSpec((_BLK, _BATCH), lambda i: (i, 0)),
            pl.BlockSpec((_BATCH, _NUM_POST), lambda i: (0, 0)),
            pl.BlockSpec((_BLK, _NUM_POST), lambda i: (i, 0)),
        ],
        out_specs=(
            pl.BlockSpec((_BLK, _NUM_POST), lambda i: (i, 0)),
            pl.BlockSpec((_BLK, _NUM_POST), lambda i: (i, 0)),
            pl.BlockSpec((1, 128), lambda i: (0, 0)),
        ),
        scratch_shapes=[pltpu.SMEM((8,), jnp.int32)],
        compiler_params=pltpu.CompilerParams(
            dimension_semantics=("arbitrary",)),
    )(scalars, pre_t, post, elig)


def _sc_body(aux_hbm, abits_hbm, comp_hbm, counts_hbm,
             data_v, comp_v, cnt_v, aux_v):
    wid = lax.axis_index("s") * 2 + lax.axis_index("c")
    lane = jax.lax.iota(jnp.int32, 16)

    pltpu.sync_copy(aux_hbm, aux_v)
    av = aux_v[...]
    t_lo = jnp.max(jnp.where(lane == 0, av, _IMIN))
    t_hi = jnp.max(jnp.where(lane == 1, av, _IMIN))

    pltpu.sync_copy(abits_hbm.at[wid], data_v)

    def step(i, off):
        v = data_v[pl.ds(i * 16, 16)]
        m = jnp.logical_and(v >= t_lo, v < t_hi)
        c = jnp.max(plsc.all_reduce_population_count(m))

        @pl.when(off <= _CAP - 16)
        def _():
            plsc.store_compressed(comp_v.at[pl.ds(off, 16)], v, mask=m)

        return off + c

    n_in = lax.fori_loop(0, _PER_W // 16, step, jnp.int32(0))

    cnt_v[...] = jnp.where(lane == 0, n_in, 0)
    pltpu.sync_copy(cnt_v, counts_hbm.at[wid])
    pltpu.sync_copy(comp_v, comp_hbm.at[wid])


def _sc_compact(aux, abits):
    mesh = plsc.VectorSubcoreMesh(core_axis_name="c", subcore_axis_name="s")
    return pl.kernel(
        _sc_body,
        out_type=(
            jax.ShapeDtypeStruct((_NW, _CAP), jnp.int32),
            jax.ShapeDtypeStruct((_NW, 16), jnp.int32),
        ),
        mesh=mesh,
        scratch_types=[
            pltpu.VMEM((_PER_W,), jnp.int32),
            pltpu.VMEM((_CAP,), jnp.int32),
            pltpu.VMEM((16,), jnp.int32),
            pltpu.VMEM((16,), jnp.int32),
        ],
        compiler_params=pltpu.CompilerParams(needs_layout_passes=False),
    )(aux, abits)


def _tc2a_body(scal_ref, iscal_ref, comp_ref, counts_ref, ne_blk_ref,
               wu_ref, smem):
    i = pl.program_id(0)
    mwc = scal_ref[3]

    @pl.when(i == 0)
    def _search():
        cut_else = iscal_ref[0]
        use_topk = iscal_ref[1]
        n_above = iscal_ref[2]
        t_lo = iscal_ref[3]
        t_hi = iscal_ref[4]
        n_in = iscal_ref[5]
        comp = comp_ref[...]
        n_i = counts_ref[:, 0:1]
        col = jax.lax.broadcasted_iota(jnp.int32, (_NW, _CAP), 1)
        valid = col < n_i

        def count_fn(mid):
            sel = jnp.logical_and(valid, comp >= mid)
            return n_above + jnp.sum(sel.astype(jnp.int32))

        tstar = _count_bisect(count_fn, t_lo, t_hi, n_above + n_in)
        smem[0] = jnp.where(use_topk > 0, tstar, cut_else)

    cut = smem[0]
    ne = ne_blk_ref[...]
    abits = jax.lax.bitcast_convert_type(jnp.abs(ne), jnp.int32)
    wu_ref[...] = jnp.where(abits >= cut, jnp.clip(ne, -mwc, mwc),
                            jnp.zeros_like(ne))


def _tc2a(scalars, iscal, comp, counts, ne):
    return pl.pallas_call(
        _tc2a_body,
        grid=(_G,),
        out_shape=jax.ShapeDtypeStruct((_NUM_PRE, _NUM_POST), jnp.float32),
        in_specs=[
            pl.BlockSpec(memory_space=pltpu.SMEM),
            pl.BlockSpec(memory_space=pltpu.SMEM),
            pl.BlockSpec((_NW, _CAP), lambda i: (0, 0)),
            pl.BlockSpec((_NW, 16), lambda i: (0, 0)),
            pl.BlockSpec((_BLK, _NUM_POST), lambda i: (i, 0)),
        ],
        out_specs=pl.BlockSpec((_BLK, _NUM_POST), lambda i: (i, 0)),
        scratch_shapes=[pltpu.SMEM((2,), jnp.int32)],
        compiler_params=pltpu.CompilerParams(
            dimension_semantics=("arbitrary",)),
    )(scalars, iscal, comp, counts, ne)


def _tc2b_body(scal_ref, iscal_ref, ne_blk_ref, wu_ref, bits_scr, smem):
    i = pl.program_id(0)
    mwc = scal_ref[3]

    @pl.when(i < _G)
    def _stage():
        ne = ne_blk_ref[...]
        bits_scr[pl.ds(i * _BLK, _BLK), :] = jax.lax.bitcast_convert_type(
            jnp.abs(ne), jnp.int32)

    @pl.when(i == _G)
    def _search():
        bits = bits_scr[...]

        def count_fn(mid):
            return jnp.sum((bits >= mid).astype(jnp.int32))

        tstar = _count_bisect(count_fn, jnp.int32(0), jnp.int32(_INF_BITS),
                              jnp.int32(_N))
        smem[0] = jnp.where(iscal_ref[1] > 0, tstar, iscal_ref[0])

    @pl.when(i >= _G)
    def _write():
        j = i - _G
        cut = smem[0]
        ne = ne_blk_ref[...]
        abits = jax.lax.bitcast_convert_type(jnp.abs(ne), jnp.int32)
        wu_ref[...] = jnp.where(abits >= cut, jnp.clip(ne, -mwc, mwc),
                                jnp.zeros_like(ne))


def _tc2b(scalars, iscal, ne):
    return pl.pallas_call(
        _tc2b_body,
        grid=(2 * _G,),
        out_shape=jax.ShapeDtypeStruct((_NUM_PRE, _NUM_POST), jnp.float32),
        in_specs=[
            pl.BlockSpec(memory_space=pltpu.SMEM),
            pl.BlockSpec(memory_space=pltpu.SMEM),
            pl.BlockSpec((_BLK, _NUM_POST),
                         lambda i: (jnp.where(i < _G, i, i - _G), 0)),
        ],
        out_specs=pl.BlockSpec((_BLK, _NUM_POST),
                               lambda i: (jnp.maximum(i - _G, 0), 0)),
        scratch_shapes=[
            pltpu.VMEM((_NUM_PRE, _NUM_POST), jnp.int32),
            pltpu.SMEM((2,), jnp.int32),
        ],
        compiler_params=pltpu.CompilerParams(
            dimension_semantics=("arbitrary",)),
    )(scalars, iscal, ne)


def kernel(pre_spikes, post_spikes, weights, eligibility_trace, a_plus,
           a_minus, tau_plus, tau_minus, tau_eligibility, activity_threshold,
           max_weight_change):
    del weights, tau_plus, tau_minus
    decay = jnp.exp(-_DT / tau_eligibility)
    scale = (a_plus + a_minus) / _BATCH
    scalars = jnp.stack([decay, scale, activity_threshold,
                         max_weight_change]).astype(jnp.float32)
    pre_t = pre_spikes.T

    ne, abits, meta = _tc1(scalars, pre_t, post_spikes, eligibility_trace)

    t_lo = meta[0, 0]
    t_hi = meta[0, 1]
    cnt_thr = meta[0, 2]
    n_above = meta[0, 3]
    n_in = meta[0, 4]

    aux = jnp.zeros((16,), jnp.int32).at[0].set(t_lo).at[1].set(t_hi)
    comp, counts = _sc_compact(aux, abits.reshape(_NW, _PER_W))

    thr_bits = jax.lax.bitcast_convert_type(activity_threshold, jnp.int32)
    cut_else = jnp.where(activity_threshold >= 0.0, thr_bits + 1,
                         jnp.int32(0))
    use_topk = (cnt_thr > _K_TARGET).astype(jnp.int32)
    iscal = jnp.stack([cut_else, use_topk, n_above, t_lo, t_hi, n_in,
                       jnp.int32(0), jnp.int32(0)])

    overflow = jnp.any(counts[:, 0] > _CAP - 16)
    ok = jnp.logical_and(
        jnp.logical_not(overflow),
        jnp.logical_and(n_above < _K_TARGET,
                        n_above + n_in >= _K_TARGET))

    wu = lax.cond(
        ok,
        lambda: _tc2a(scalars, iscal, comp, counts, ne),
        lambda: _tc2b(scalars, iscal, ne),
    )
    return (wu, ne)


# speedup vs baseline: 1.1089x; 1.1089x over previous
import jax
import jax.numpy as jnp
from jax import lax
from jax.experimental import pallas as pl
from jax.experimental.pallas import tpu as pltpu
from jax.experimental.pallas import tpu_sc as plsc

_NUM_PRE = 2048
_NUM_POST = 1024
_BATCH = 16
_N = _NUM_PRE * _NUM_POST
_K_TARGET = int(0.1 * _N)
_DT = 0.1
_BLK = 256
_G = _NUM_PRE // _BLK

_SUB_ROWS = 64
_K_SUB = _K_TARGET // 32
_M_SUB = 480
_SUB_ITERS = 22
_RANK_TOL = 3
_INF_BITS = 0x7F800000

_NW = 32
_PER_W = _N // _NW
_NSEG = 4
_CAP = 4096
_SEG_CAP = _CAP // _NSEG
_IMIN = jnp.iinfo(jnp.int32).min


def _sub_bisect(data, target, n_iter):
    def step(_, lohi):
        lo, hi = lohi
        mid = lo + (hi - lo) // 2
        c = jnp.sum((data >= mid).astype(jnp.int32))
        ge = c >= target
        return jnp.where(ge, mid, lo), jnp.where(ge, hi, mid)

    lo, _ = lax.fori_loop(0, n_iter, step, (jnp.int32(0),
                                            jnp.int32(_INF_BITS)))
    return lo


def _count_bisect(count_fn, lo0, hi0, c0):
    def cond(state):
        lo, hi, c_lo = state
        return jnp.logical_and(c_lo > _K_TARGET + _RANK_TOL, hi - lo > 1)

    def body(state):
        lo, hi, c_lo = state
        mid = lo + (hi - lo) // 2
        c = count_fn(mid)
        ge = c >= _K_TARGET
        return (jnp.where(ge, mid, lo), jnp.where(ge, hi, mid),
                jnp.where(ge, c, c_lo))

    lo, _, _ = lax.while_loop(cond, body, (lo0, hi0, c0))
    return lo


def _tc1_body(scal_ref, pre_blk_ref, post_ref, elig_blk_ref,
              ne_ref, abits_ref, meta_ref, smem):
    i = pl.program_id(0)
    decay = scal_ref[0]
    scale = scal_ref[1]
    thr = scal_ref[2]

    upd = jnp.dot(pre_blk_ref[...], post_ref[...],
                  preferred_element_type=jnp.float32) * scale
    ne = elig_blk_ref[...] * decay + upd
    ne_ref[...] = ne
    abits = jax.lax.bitcast_convert_type(jnp.abs(ne), jnp.int32)
    abits_ref[...] = abits
    cnt = jnp.sum((jnp.abs(ne) > thr).astype(jnp.int32))

    @pl.when(i == 0)
    def _first():
        sub = abits[0:_SUB_ROWS, :]
        t_lo = _sub_bisect(sub, _K_SUB + _M_SUB, _SUB_ITERS)
        t_hi = _sub_bisect(sub, _K_SUB - _M_SUB, _SUB_ITERS)
        smem[0] = cnt
        smem[1] = jnp.sum((abits >= t_hi).astype(jnp.int32))
        smem[2] = jnp.sum(jnp.logical_and(abits >= t_lo,
                                          abits < t_hi).astype(jnp.int32))
        smem[3] = t_lo
        smem[4] = t_hi

    @pl.when(i > 0)
    def _rest():
        t_lo = smem[3]
        t_hi = smem[4]
        smem[0] = smem[0] + cnt
        smem[1] = smem[1] + jnp.sum((abits >= t_hi).astype(jnp.int32))
        smem[2] = smem[2] + jnp.sum(
            jnp.logical_and(abits >= t_lo, abits < t_hi).astype(jnp.int32))

    @pl.when(i == _G - 1)
    def _last():
        ci = jax.lax.broadcasted_iota(jnp.int32, (1, 128), 1)
        meta_ref[...] = jnp.where(
            ci == 0, smem[3],
            jnp.where(ci == 1, smem[4],
                      jnp.where(ci == 2, smem[0],
                                jnp.where(ci == 3, smem[1],
                                          jnp.where(ci == 4, smem[2], 0)))))


def _tc1(scalars, pre_t, post, elig):
    out_shape = (
        jax.ShapeDtypeStruct((_NUM_PRE, _NUM_POST), jnp.float32),
        jax.ShapeDtypeStruct((_NUM_PRE, _NUM_POST), jnp.int32),
        jax.ShapeDtypeStruct((1, 128), jnp.int32),
    )
    return pl.pallas_call(
        _tc1_body,
        grid=(_G,),
        out_shape=out_shape,
        in_specs=[
            pl.BlockSpec(memory_space=pltpu.SMEM),
            pl.BlockSpec((_BLK, _BATCH), lambda i: (i, 0)),
            pl.BlockSpec((_BATCH, _NUM_POST), lambda i: (0, 0)),
            pl.BlockSpec((_BLK, _NUM_POST), lambda i: (i, 0)),
        ],
        out_specs=(
            pl.BlockSpec((_BLK, _NUM_POST), lambda i: (i, 0)),
            pl.BlockSpec((_BLK, _NUM_POST), lambda i: (i, 0)),
            pl.BlockSpec((1, 128), lambda i: (0, 0)),
        ),
        scratch_shapes=[pltpu.SMEM((8,), jnp.int32)],
        compiler_params=pltpu.CompilerParams(
            dimension_semantics=("arbitrary",)),
    )(scalars, pre_t, post, elig)


def _sc_body(aux_hbm, abits_hbm, comp_hbm, counts_hbm,
             data_v, comp_v, cnt_v, aux_v):
    wid = lax.axis_index("s") * 2 + lax.axis_index("c")
    lane = jax.lax.iota(jnp.int32, 16)

    pltpu.sync_copy(aux_hbm, aux_v)
    av = aux_v[...]
    t_lo = jnp.max(jnp.where(lane == 0, av, _IMIN))
    t_hi = jnp.max(jnp.where(lane == 1, av, _IMIN))

    pltpu.sync_copy(abits_hbm.at[wid], data_v)

    n_iter = _PER_W // 16 // _NSEG

    def step(i, offs):
        new = []
        for s in range(_NSEG):
            v = data_v[pl.ds((s * n_iter + i) * 16, 16)]
            m = jnp.logical_and(v >= t_lo, v < t_hi)
            c = jnp.max(plsc.all_reduce_population_count(m))
            off = offs[s]

            @pl.when(off <= _SEG_CAP - 16)
            def _(s=s, off=off, v=v, m=m):
                plsc.store_compressed(
                    comp_v.at[pl.ds(s * _SEG_CAP + off, 16)], v, mask=m)

            new.append(off + c)
        return tuple(new)

    offs = lax.fori_loop(0, n_iter, step,
                         tuple(jnp.int32(0) for _ in range(_NSEG)))

    cnt = jnp.full((16,), 0, jnp.int32)
    for s in range(_NSEG):
        cnt = jnp.where(lane == s, offs[s], cnt)
    cnt_v[...] = cnt
    pltpu.sync_copy(cnt_v, counts_hbm.at[wid])
    pltpu.sync_copy(comp_v, comp_hbm.at[wid])


def _sc_compact(aux, abits):
    mesh = plsc.VectorSubcoreMesh(core_axis_name="c", subcore_axis_name="s")
    return pl.kernel(
        _sc_body,
        out_type=(
            jax.ShapeDtypeStruct((_NW, _CAP), jnp.int32),
            jax.ShapeDtypeStruct((_NW, 16), jnp.int32),
        ),
        mesh=mesh,
        scratch_types=[
            pltpu.VMEM((_PER_W,), jnp.int32),
            pltpu.VMEM((_CAP,), jnp.int32),
            pltpu.VMEM((16,), jnp.int32),
            pltpu.VMEM((16,), jnp.int32),
        ],
        compiler_params=pltpu.CompilerParams(needs_layout_passes=False),
    )(aux, abits)


def _tc2a_body(scal_ref, iscal_ref, comp_ref, counts_ref, ne_blk_ref,
               wu_ref, smem):
    i = pl.program_id(0)
    mwc = scal_ref[3]

    @pl.when(i == 0)
    def _search():
        cut_else = iscal_ref[0]
        use_topk = iscal_ref[1]
        n_above = iscal_ref[2]
        t_lo = iscal_ref[3]
        t_hi = iscal_ref[4]
        n_in = iscal_ref[5]
        comp = comp_ref[...]
        col = jax.lax.broadcasted_iota(jnp.int32, (_NW, _CAP), 1)
        valid = jnp.zeros((_NW, _CAP), jnp.bool_)
        for s in range(_NSEG):
            n_s = counts_ref[:, s:s + 1]
            valid = jnp.logical_or(
                valid,
                jnp.logical_and(col >= s * _SEG_CAP,
                                col - s * _SEG_CAP < n_s))

        def count_fn(mid):
            sel = jnp.logical_and(valid, comp >= mid)
            return n_above + jnp.sum(sel.astype(jnp.int32))

        tstar = _count_bisect(count_fn, t_lo, t_hi, n_above + n_in)
        smem[0] = jnp.where(use_topk > 0, tstar, cut_else)

    cut = smem[0]
    ne = ne_blk_ref[...]
    abits = jax.lax.bitcast_convert_type(jnp.abs(ne), jnp.int32)
    wu_ref[...] = jnp.where(abits >= cut, jnp.clip(ne, -mwc, mwc),
                            jnp.zeros_like(ne))


def _tc2a(scalars, iscal, comp, counts, ne):
    return pl.pallas_call(
        _tc2a_body,
        grid=(_G,),
        out_shape=jax.ShapeDtypeStruct((_NUM_PRE, _NUM_POST), jnp.float32),
        in_specs=[
            pl.BlockSpec(memory_space=pltpu.SMEM),
            pl.BlockSpec(memory_space=pltpu.SMEM),
            pl.BlockSpec((_NW, _CAP), lambda i: (0, 0)),
            pl.BlockSpec((_NW, 16), lambda i: (0, 0)),
            pl.BlockSpec((_BLK, _NUM_POST), lambda i: (i, 0)),
        ],
        out_specs=pl.BlockSpec((_BLK, _NUM_POST), lambda i: (i, 0)),
        scratch_shapes=[pltpu.SMEM((2,), jnp.int32)],
        compiler_params=pltpu.CompilerParams(
            dimension_semantics=("arbitrary",)),
    )(scalars, iscal, comp, counts, ne)


def _tc2b_body(scal_ref, iscal_ref, ne_blk_ref, wu_ref, bits_scr, smem):
    i = pl.program_id(0)
    mwc = scal_ref[3]

    @pl.when(i < _G)
    def _stage():
        ne = ne_blk_ref[...]
        bits_scr[pl.ds(i * _BLK, _BLK), :] = jax.lax.bitcast_convert_type(
            jnp.abs(ne), jnp.int32)

    @pl.when(i == _G)
    def _search():
        bits = bits_scr[...]

        def count_fn(mid):
            return jnp.sum((bits >= mid).astype(jnp.int32))

        tstar = _count_bisect(count_fn, jnp.int32(0), jnp.int32(_INF_BITS),
                              jnp.int32(_N))
        smem[0] = jnp.where(iscal_ref[1] > 0, tstar, iscal_ref[0])

    @pl.when(i >= _G)
    def _write():
        j = i - _G
        cut = smem[0]
        ne = ne_blk_ref[...]
        abits = jax.lax.bitcast_convert_type(jnp.abs(ne), jnp.int32)
        wu_ref[...] = jnp.where(abits >= cut, jnp.clip(ne, -mwc, mwc),
                                jnp.zeros_like(ne))


def _tc2b(scalars, iscal, ne):
    return pl.pallas_call(
        _tc2b_body,
        grid=(2 * _G,),
        out_shape=jax.ShapeDtypeStruct((_NUM_PRE, _NUM_POST), jnp.float32),
        in_specs=[
            pl.BlockSpec(memory_space=pltpu.SMEM),
            pl.BlockSpec(memory_space=pltpu.SMEM),
            pl.BlockSpec((_BLK, _NUM_POST),
                         lambda i: (jnp.where(i < _G, i, i - _G), 0)),
        ],
        out_specs=pl.BlockSpec((_BLK, _NUM_POST),
                               lambda i: (jnp.maximum(i - _G, 0), 0)),
        scratch_shapes=[
            pltpu.VMEM((_NUM_PRE, _NUM_POST), jnp.int32),
            pltpu.SMEM((2,), jnp.int32),
        ],
        compiler_params=pltpu.CompilerParams(
            dimension_semantics=("arbitrary",)),
    )(scalars, iscal, ne)


def kernel(pre_spikes, post_spikes, weights, eligibility_trace, a_plus,
           a_minus, tau_plus, tau_minus, tau_eligibility, activity_threshold,
           max_weight_change):
    del weights, tau_plus, tau_minus
    decay = jnp.exp(-_DT / tau_eligibility)
    scale = (a_plus + a_minus) / _BATCH
    scalars = jnp.stack([decay, scale, activity_threshold,
                         max_weight_change]).astype(jnp.float32)
    pre_t = pre_spikes.T

    ne, abits, meta = _tc1(scalars, pre_t, post_spikes, eligibility_trace)

    t_lo = meta[0, 0]
    t_hi = meta[0, 1]
    cnt_thr = meta[0, 2]
    n_above = meta[0, 3]
    n_in = meta[0, 4]

    aux = jnp.zeros((16,), jnp.int32).at[0].set(t_lo).at[1].set(t_hi)
    comp, counts = _sc_compact(aux, abits.reshape(_NW, _PER_W))

    thr_bits = jax.lax.bitcast_convert_type(activity_threshold, jnp.int32)
    cut_else = jnp.where(activity_threshold >= 0.0, thr_bits + 1,
                         jnp.int32(0))
    use_topk = (cnt_thr > _K_TARGET).astype(jnp.int32)
    iscal = jnp.stack([cut_else, use_topk, n_above, t_lo, t_hi, n_in,
                       jnp.int32(0), jnp.int32(0)])

    overflow = jnp.any(counts[:, 0:_NSEG] > _SEG_CAP - 16)
    ok = jnp.logical_and(
        jnp.logical_not(overflow),
        jnp.logical_and(n_above < _K_TARGET,
                        n_above + n_in >= _K_TARGET))

    wu = lax.cond(
        ok,
        lambda: _tc2a(scalars, iscal, comp, counts, ne),
        lambda: _tc2b(scalars, iscal, ne),
    )
    return (wu, ne)


# speedup vs baseline: 2.0471x; 1.8462x over previous
import jax
import jax.numpy as jnp
from jax import lax
from jax.experimental import pallas as pl
from jax.experimental.pallas import tpu as pltpu

_NUM_PRE = 2048
_NUM_POST = 1024
_BATCH = 16
_N = _NUM_PRE * _NUM_POST
_K_TARGET = int(0.1 * _N)
_DT = 0.1
_BLK = 256
_G = _NUM_PRE // _BLK

_SUB_ROWS = 64
_K_SUB = _K_TARGET // 32
_M_SUB = 480
_SUB_ITERS = 22
_RANK_TOL = 6
_INF_BITS = 0x7F800000


def _sub_bisect(data, target, n_iter):
    def step(_, lohi):
        lo, hi = lohi
        mid = lo + (hi - lo) // 2
        c = jnp.sum((data >= mid).astype(jnp.int32))
        ge = c >= target
        return jnp.where(ge, mid, lo), jnp.where(ge, hi, mid)

    lo, _ = lax.fori_loop(0, n_iter, step, (jnp.int32(0),
                                            jnp.int32(_INF_BITS)))
    return lo


def _body(scal_ref, pre_blk_ref, post_ref, elig_blk_ref, wu_ref, elig_out_ref,
          ne_scr, bits_scr, smem):
    i = pl.program_id(0)
    decay = scal_ref[0]
    scale = scal_ref[1]
    thr = scal_ref[2]
    mwc = scal_ref[3]

    @pl.when(i < _G)
    def _phase_a():
        upd = jnp.dot(pre_blk_ref[...], post_ref[...],
                      preferred_element_type=jnp.float32) * scale
        ne = elig_blk_ref[...] * decay + upd
        elig_out_ref[...] = ne
        act = jnp.abs(ne)
        abits = jax.lax.bitcast_convert_type(act, jnp.int32)
        ne_scr[pl.ds(i * _BLK, _BLK), :] = ne
        bits_scr[pl.ds(i * _BLK, _BLK), :] = abits
        cnt = jnp.sum((act > thr).astype(jnp.int32))

        @pl.when(i == 0)
        def _first():
            sub = abits[0:_SUB_ROWS, :]
            t_lo = _sub_bisect(sub, _K_SUB + _M_SUB, _SUB_ITERS)
            t_hi = _sub_bisect(sub, _K_SUB - _M_SUB, _SUB_ITERS)
            smem[0] = cnt
            smem[1] = jnp.sum((abits >= t_hi).astype(jnp.int32))
            smem[2] = jnp.sum(jnp.logical_and(
                abits >= t_lo, abits < t_hi).astype(jnp.int32))
            smem[3] = t_lo
            smem[4] = t_hi

        @pl.when(i > 0)
        def _rest():
            t_lo = smem[3]
            t_hi = smem[4]
            smem[0] = smem[0] + cnt
            smem[1] = smem[1] + jnp.sum((abits >= t_hi).astype(jnp.int32))
            smem[2] = smem[2] + jnp.sum(jnp.logical_and(
                abits >= t_lo, abits < t_hi).astype(jnp.int32))

    @pl.when(i == _G)
    def _phase_b():
        num_updates = smem[0]
        n_above = smem[1]
        n_in = smem[2]
        t_lo = smem[3]
        t_hi = smem[4]

        ok = jnp.logical_and(n_above < _K_TARGET,
                             n_above + n_in >= _K_TARGET)
        lo0 = jnp.where(ok, t_lo, jnp.int32(0))
        hi0 = jnp.where(ok, t_hi, jnp.int32(_INF_BITS))
        c0 = jnp.where(ok, n_above + n_in, jnp.int32(_N))

        bits = bits_scr[...]

        def cond(state):
            lo, hi, c_lo = state
            return jnp.logical_and(c_lo > _K_TARGET + _RANK_TOL, hi - lo > 1)

        def body(state):
            lo, hi, c_lo = state
            mid = lo + (hi - lo) // 2
            c = jnp.sum((bits >= mid).astype(jnp.int32))
            ge = c >= _K_TARGET
            return (jnp.where(ge, mid, lo), jnp.where(ge, hi, mid),
                    jnp.where(ge, c, c_lo))

        tstar, _, _ = lax.while_loop(cond, body, (lo0, hi0, c0))

        use_topk = num_updates > _K_TARGET
        thr_bits = jax.lax.bitcast_convert_type(thr, jnp.int32)
        thr_cut = jnp.where(thr >= 0.0, thr_bits + 1, jnp.int32(0))
        smem[5] = jnp.where(use_topk, tstar, thr_cut)

    @pl.when(i >= _G)
    def _phase_c():
        j = i - _G
        cut = smem[5]
        ne = ne_scr[pl.ds(j * _BLK, _BLK), :]
        abits = bits_scr[pl.ds(j * _BLK, _BLK), :]
        mask = abits >= cut
        wu_ref[...] = jnp.where(mask, jnp.clip(ne, -mwc, mwc),
                                jnp.zeros_like(ne))


def kernel(pre_spikes, post_spikes, weights, eligibility_trace, a_plus,
           a_minus, tau_plus, tau_minus, tau_eligibility, activity_threshold,
           max_weight_change):
    del weights, tau_plus, tau_minus
    decay = jnp.exp(-_DT / tau_eligibility)
    scale = (a_plus + a_minus) / _BATCH
    scalars = jnp.stack([decay, scale, activity_threshold,
                         max_weight_change]).astype(jnp.float32)
    pre_t = pre_spikes.T

    out_shape = (
        jax.ShapeDtypeStruct((_NUM_PRE, _NUM_POST), jnp.float32),
        jax.ShapeDtypeStruct((_NUM_PRE, _NUM_POST), jnp.float32),
    )
    wu, new_elig = pl.pallas_call(
        _body,
        grid=(2 * _G,),
        out_shape=out_shape,
        in_specs=[
            pl.BlockSpec(memory_space=pltpu.SMEM),
            pl.BlockSpec((_BLK, _BATCH),
                         lambda i: (jnp.minimum(i, _G - 1), 0)),
            pl.BlockSpec((_BATCH, _NUM_POST), lambda i: (0, 0)),
            pl.BlockSpec((_BLK, _NUM_POST),
                         lambda i: (jnp.minimum(i, _G - 1), 0)),
        ],
        out_specs=(
            pl.BlockSpec((_BLK, _NUM_POST),
                         lambda i: (jnp.maximum(i - _G, 0), 0)),
            pl.BlockSpec((_BLK, _NUM_POST),
                         lambda i: (jnp.minimum(i, _G - 1), 0)),
        ),
        scratch_shapes=[
            pltpu.VMEM((_NUM_PRE, _NUM_POST), jnp.float32),
            pltpu.VMEM((_NUM_PRE, _NUM_POST), jnp.int32),
            pltpu.SMEM((8,), jnp.int32),
        ],
        compiler_params=pltpu.CompilerParams(
            dimension_semantics=("arbitrary",)),
    )(scalars, pre_t, post_spikes, eligibility_trace)
    return (wu, new_elig)


# speedup vs baseline: 2.9249x; 1.4288x over previous
import jax
import jax.numpy as jnp
from jax import lax
from jax.experimental import pallas as pl
from jax.experimental.pallas import tpu as pltpu

_NUM_PRE = 2048
_NUM_POST = 1024
_BATCH = 16
_N = _NUM_PRE * _NUM_POST
_K_TARGET = int(0.1 * _N)
_DT = 0.1
_BLK = 256
_G = _NUM_PRE // _BLK

_SUB_ROWS = 64
_K_SUB = _K_TARGET // 32
_M_SUB = 480
_SUB_ITERS = 22
_RANK_TOL = 6
_INF_BITS = 0x7F800000


def _sub_bisect(data, target, n_iter):
    def step(_, lohi):
        lo, hi = lohi
        mid = lo + (hi - lo) // 2
        c = jnp.sum((data >= mid).astype(jnp.int32))
        ge = c >= target
        return jnp.where(ge, mid, lo), jnp.where(ge, hi, mid)

    lo, _ = lax.fori_loop(0, n_iter, step, (jnp.int32(0),
                                            jnp.int32(_INF_BITS)))
    return lo


def _body(scal_ref, pre_blk_ref, post_ref, elig_blk_ref, wu_ref, elig_out_ref,
          ne_scr, bits_scr, smem):
    i = pl.program_id(0)
    decay = scal_ref[0]
    scale = scal_ref[1]
    thr = scal_ref[2]
    mwc = scal_ref[3]

    @pl.when(i < _G)
    def _phase_a():
        upd = jnp.dot(pre_blk_ref[...], post_ref[...],
                      preferred_element_type=jnp.float32) * scale
        ne = elig_blk_ref[...] * decay + upd
        elig_out_ref[...] = ne
        act = jnp.abs(ne)
        abits = jax.lax.bitcast_convert_type(act, jnp.int32)
        ne_scr[pl.ds(i * _BLK, _BLK), :] = ne
        bits_scr[pl.ds(i * _BLK, _BLK), :] = abits
        cnt = jnp.sum((act > thr).astype(jnp.int32))

        @pl.when(i == 0)
        def _first():
            sub = abits[0:_SUB_ROWS, :]
            t_lo = _sub_bisect(sub, _K_SUB + _M_SUB, _SUB_ITERS)
            t_hi = _sub_bisect(sub, _K_SUB - _M_SUB, _SUB_ITERS)
            smem[0] = cnt
            smem[1] = jnp.sum((abits >= t_hi).astype(jnp.int32))
            smem[2] = jnp.sum(jnp.logical_and(
                abits >= t_lo, abits < t_hi).astype(jnp.int32))
            smem[3] = t_lo
            smem[4] = t_hi

        @pl.when(i > 0)
        def _rest():
            t_lo = smem[3]
            t_hi = smem[4]
            smem[0] = smem[0] + cnt
            smem[1] = smem[1] + jnp.sum((abits >= t_hi).astype(jnp.int32))
            smem[2] = smem[2] + jnp.sum(jnp.logical_and(
                abits >= t_lo, abits < t_hi).astype(jnp.int32))

    @pl.when(i == _G)
    def _phase_b():
        num_updates = smem[0]
        n_above = smem[1]
        n_in = smem[2]
        t_lo = smem[3]
        t_hi = smem[4]

        ok = jnp.logical_and(n_above < _K_TARGET,
                             n_above + n_in >= _K_TARGET)
        lo0 = jnp.where(ok, t_lo, jnp.int32(0))
        hi0 = jnp.where(ok, t_hi, jnp.int32(_INF_BITS))
        c0 = jnp.where(ok, n_above + n_in, jnp.int32(_N))

        bits = bits_scr[...]
        ones_row = jnp.ones((1, _NUM_PRE), jnp.float32)

        def cond(state):
            lo, hi, c_lo = state
            return jnp.logical_and(c_lo > _K_TARGET + _RANK_TOL, hi - lo > 1)

        def body(state):
            lo, hi, c_lo = state
            mid = lo + (hi - lo) // 2
            maskf = jnp.where(bits >= mid, 1.0, 0.0)
            col = jnp.dot(ones_row, maskf,
                          preferred_element_type=jnp.float32)
            c = jnp.sum(col).astype(jnp.int32)
            ge = c >= _K_TARGET
            return (jnp.where(ge, mid, lo), jnp.where(ge, hi, mid),
                    jnp.where(ge, c, c_lo))

        tstar, _, _ = lax.while_loop(cond, body, (lo0, hi0, c0))

        use_topk = num_updates > _K_TARGET
        thr_bits = jax.lax.bitcast_convert_type(thr, jnp.int32)
        thr_cut = jnp.where(thr >= 0.0, thr_bits + 1, jnp.int32(0))
        smem[5] = jnp.where(use_topk, tstar, thr_cut)

    @pl.when(i >= _G)
    def _phase_c():
        j = i - _G
        cut = smem[5]
        ne = ne_scr[pl.ds(j * _BLK, _BLK), :]
        abits = bits_scr[pl.ds(j * _BLK, _BLK), :]
        mask = abits >= cut
        wu_ref[...] = jnp.where(mask, jnp.clip(ne, -mwc, mwc),
                                jnp.zeros_like(ne))


def kernel(pre_spikes, post_spikes, weights, eligibility_trace, a_plus,
           a_minus, tau_plus, tau_minus, tau_eligibility, activity_threshold,
           max_weight_change):
    del weights, tau_plus, tau_minus
    decay = jnp.exp(-_DT / tau_eligibility)
    scale = (a_plus + a_minus) / _BATCH
    scalars = jnp.stack([decay, scale, activity_threshold,
                         max_weight_change]).astype(jnp.float32)
    pre_t = pre_spikes.T

    out_shape = (
        jax.ShapeDtypeStruct((_NUM_PRE, _NUM_POST), jnp.float32),
        jax.ShapeDtypeStruct((_NUM_PRE, _NUM_POST), jnp.float32),
    )
    wu, new_elig = pl.pallas_call(
        _body,
        grid=(2 * _G,),
        out_shape=out_shape,
        in_specs=[
            pl.BlockSpec(memory_space=pltpu.SMEM),
            pl.BlockSpec((_BLK, _BATCH),
                         lambda i: (jnp.minimum(i, _G - 1), 0)),
            pl.BlockSpec((_BATCH, _NUM_POST), lambda i: (0, 0)),
            pl.BlockSpec((_BLK, _NUM_POST),
                         lambda i: (jnp.minimum(i, _G - 1), 0)),
        ],
        out_specs=(
            pl.BlockSpec((_BLK, _NUM_POST),
                         lambda i: (jnp.maximum(i - _G, 0), 0)),
            pl.BlockSpec((_BLK, _NUM_POST),
                         lambda i: (jnp.minimum(i, _G - 1), 0)),
        ),
        scratch_shapes=[
            pltpu.VMEM((_NUM_PRE, _NUM_POST), jnp.float32),
            pltpu.VMEM((_NUM_PRE, _NUM_POST), jnp.int32),
            pltpu.SMEM((8,), jnp.int32),
        ],
        compiler_params=pltpu.CompilerParams(
            dimension_semantics=("arbitrary",)),
    )(scalars, pre_t, post_spikes, eligibility_trace)
    return (wu, new_elig)


# speedup vs baseline: 2.9825x; 1.0197x over previous
import jax
import jax.numpy as jnp
from jax import lax
from jax.experimental import pallas as pl
from jax.experimental.pallas import tpu as pltpu

_NUM_PRE = 2048
_NUM_POST = 1024
_BATCH = 16
_N = _NUM_PRE * _NUM_POST
_K_TARGET = int(0.1 * _N)
_DT = 0.1
_BLK = 256
_G = _NUM_PRE // _BLK

_SUB_ROWS = 64
_K_SUB = _K_TARGET // 32
_M_SUB = 480
_SUB_ITERS = 22
_RANK_TOL = 6
_INF_BITS = 0x7F800000


def _sub_bisect(data, target, n_iter):
    def step(_, lohi):
        lo, hi = lohi
        mid = lo + (hi - lo) // 2
        c = jnp.sum((data >= mid).astype(jnp.int32))
        ge = c >= target
        return jnp.where(ge, mid, lo), jnp.where(ge, hi, mid)

    lo, _ = lax.fori_loop(0, n_iter, step, (jnp.int32(0),
                                            jnp.int32(_INF_BITS)))
    return lo


def _body(scal_ref, pre_blk_ref, post_ref, elig_blk_ref, wu_ref, elig_out_ref,
          ne_scr, bits_scr, smem):
    i = pl.program_id(0)
    decay = scal_ref[0]
    scale = scal_ref[1]
    thr = scal_ref[2]
    mwc = scal_ref[3]

    thr_bits = jax.lax.bitcast_convert_type(thr, jnp.int32)
    thr_cut = jnp.where(thr >= 0.0, thr_bits + 1, jnp.int32(0))

    @pl.when(i < _G)
    def _phase_a():
        upd = jnp.dot(pre_blk_ref[...], post_ref[...],
                      preferred_element_type=jnp.float32) * scale
        ne = elig_blk_ref[...] * decay + upd
        elig_out_ref[...] = ne
        act = jnp.abs(ne)
        abits = jax.lax.bitcast_convert_type(act, jnp.int32)
        ne_scr[pl.ds(i * _BLK, _BLK), :] = ne
        bits_scr[pl.ds(i * _BLK, _BLK), :] = abits

        ones_blk = jnp.ones((1, _BLK), jnp.float32)

        def blk_count(t):
            maskf = jnp.where(abits >= t, 1.0, 0.0)
            col = jnp.dot(ones_blk, maskf,
                          preferred_element_type=jnp.float32)
            return jnp.sum(col).astype(jnp.int32)

        @pl.when(i == 0)
        def _first():
            sub = abits[0:_SUB_ROWS, :]
            t_lo = _sub_bisect(sub, _K_SUB + _M_SUB, _SUB_ITERS)
            t_hi = _sub_bisect(sub, _K_SUB - _M_SUB, _SUB_ITERS)
            c_hi = blk_count(t_hi)
            smem[0] = blk_count(thr_cut)
            smem[1] = c_hi
            smem[2] = blk_count(t_lo) - c_hi
            smem[3] = t_lo
            smem[4] = t_hi

        @pl.when(i > 0)
        def _rest():
            t_lo = smem[3]
            t_hi = smem[4]
            c_hi = blk_count(t_hi)
            smem[0] = smem[0] + blk_count(thr_cut)
            smem[1] = smem[1] + c_hi
            smem[2] = smem[2] + blk_count(t_lo) - c_hi

    @pl.when(i == _G)
    def _phase_b():
        num_updates = smem[0]
        n_above = smem[1]
        n_in = smem[2]
        t_lo = smem[3]
        t_hi = smem[4]

        ok = jnp.logical_and(n_above < _K_TARGET,
                             n_above + n_in >= _K_TARGET)
        lo0 = jnp.where(ok, t_lo, jnp.int32(0))
        hi0 = jnp.where(ok, t_hi, jnp.int32(_INF_BITS))
        c0 = jnp.where(ok, n_above + n_in, jnp.int32(_N))

        bits = bits_scr[...]
        ones_row = jnp.ones((1, _NUM_PRE), jnp.float32)

        def cond(state):
            lo, hi, c_lo = state
            return jnp.logical_and(c_lo > _K_TARGET + _RANK_TOL, hi - lo > 1)

        def body(state):
            lo, hi, c_lo = state
            mid = lo + (hi - lo) // 2
            maskf = jnp.where(bits >= mid, 1.0, 0.0)
            col = jnp.dot(ones_row, maskf,
                          preferred_element_type=jnp.float32)
            c = jnp.sum(col).astype(jnp.int32)
            ge = c >= _K_TARGET
            return (jnp.where(ge, mid, lo), jnp.where(ge, hi, mid),
                    jnp.where(ge, c, c_lo))

        tstar, _, _ = lax.while_loop(cond, body, (lo0, hi0, c0))

        use_topk = num_updates > _K_TARGET
        smem[5] = jnp.where(use_topk, tstar, thr_cut)

    @pl.when(i >= _G)
    def _phase_c():
        j = i - _G
        cut = smem[5]
        ne = ne_scr[pl.ds(j * _BLK, _BLK), :]
        abits = bits_scr[pl.ds(j * _BLK, _BLK), :]
        mask = abits >= cut
        wu_ref[...] = jnp.where(mask, jnp.clip(ne, -mwc, mwc),
                                jnp.zeros_like(ne))


def kernel(pre_spikes, post_spikes, weights, eligibility_trace, a_plus,
           a_minus, tau_plus, tau_minus, tau_eligibility, activity_threshold,
           max_weight_change):
    del weights, tau_plus, tau_minus
    decay = jnp.exp(-_DT / tau_eligibility)
    scale = (a_plus + a_minus) / _BATCH
    scalars = jnp.stack([decay, scale, activity_threshold,
                         max_weight_change]).astype(jnp.float32)
    pre_t = pre_spikes.T

    out_shape = (
        jax.ShapeDtypeStruct((_NUM_PRE, _NUM_POST), jnp.float32),
        jax.ShapeDtypeStruct((_NUM_PRE, _NUM_POST), jnp.float32),
    )
    wu, new_elig = pl.pallas_call(
        _body,
        grid=(2 * _G,),
        out_shape=out_shape,
        in_specs=[
            pl.BlockSpec(memory_space=pltpu.SMEM),
            pl.BlockSpec((_BLK, _BATCH),
                         lambda i: (jnp.minimum(i, _G - 1), 0)),
            pl.BlockSpec((_BATCH, _NUM_POST), lambda i: (0, 0)),
            pl.BlockSpec((_BLK, _NUM_POST),
                         lambda i: (jnp.minimum(i, _G - 1), 0)),
        ],
        out_specs=(
            pl.BlockSpec((_BLK, _NUM_POST),
                         lambda i: (jnp.maximum(i - _G, 0), 0)),
            pl.BlockSpec((_BLK, _NUM_POST),
                         lambda i: (jnp.minimum(i, _G - 1), 0)),
        ),
        scratch_shapes=[
            pltpu.VMEM((_NUM_PRE, _NUM_POST), jnp.float32),
            pltpu.VMEM((_NUM_PRE, _NUM_POST), jnp.int32),
            pltpu.SMEM((8,), jnp.int32),
        ],
        compiler_params=pltpu.CompilerParams(
            dimension_semantics=("arbitrary",)),
    )(scalars, pre_t, post_spikes, eligibility_trace)
    return (wu, new_elig)


# speedup vs baseline: 3.2551x; 1.0914x over previous
import jax
import jax.numpy as jnp
from jax import lax
from jax.experimental import pallas as pl
from jax.experimental.pallas import tpu as pltpu

_NUM_PRE = 2048
_NUM_POST = 1024
_BATCH = 16
_N = _NUM_PRE * _NUM_POST
_K_TARGET = int(0.1 * _N)
_DT = 0.1
_BLK = 512
_G = _NUM_PRE // _BLK

_SUB_ROWS = 64
_K_SUB = _K_TARGET // 32
_M_SUB = 480
_SUB_ITERS = 22
_RANK_TOL = 6
_INF_BITS = 0x7F800000


def _sub_bisect(data, target, n_iter):
    def step(_, lohi):
        lo, hi = lohi
        mid = lo + (hi - lo) // 2
        c = jnp.sum((data >= mid).astype(jnp.int32))
        ge = c >= target
        return jnp.where(ge, mid, lo), jnp.where(ge, hi, mid)

    lo, _ = lax.fori_loop(0, n_iter, step, (jnp.int32(0),
                                            jnp.int32(_INF_BITS)))
    return lo


def _body(scal_ref, pre_blk_ref, post_ref, elig_blk_ref, wu_ref, elig_out_ref,
          ne_scr, bits_scr, smem):
    i = pl.program_id(0)
    decay = scal_ref[0]
    scale = scal_ref[1]
    thr = scal_ref[2]
    mwc = scal_ref[3]

    thr_bits = jax.lax.bitcast_convert_type(thr, jnp.int32)
    thr_cut = jnp.where(thr >= 0.0, thr_bits + 1, jnp.int32(0))

    @pl.when(i < _G)
    def _phase_a():
        upd = jnp.dot(pre_blk_ref[...], post_ref[...],
                      preferred_element_type=jnp.float32) * scale
        ne = elig_blk_ref[...] * decay + upd
        elig_out_ref[...] = ne
        act = jnp.abs(ne)
        abits = jax.lax.bitcast_convert_type(act, jnp.int32)
        ne_scr[pl.ds(i * _BLK, _BLK), :] = ne
        bits_scr[pl.ds(i * _BLK, _BLK), :] = abits

        ones_blk = jnp.ones((1, _BLK), jnp.float32)

        def blk_count(t):
            maskf = jnp.where(abits >= t, 1.0, 0.0)
            col = jnp.dot(ones_blk, maskf,
                          preferred_element_type=jnp.float32)
            return jnp.sum(col).astype(jnp.int32)

        @pl.when(i == 0)
        def _first():
            sub = abits[0:_SUB_ROWS, :]
            t_lo = _sub_bisect(sub, _K_SUB + _M_SUB, _SUB_ITERS)
            t_hi = _sub_bisect(sub, _K_SUB - _M_SUB, _SUB_ITERS)
            c_hi = blk_count(t_hi)
            smem[0] = blk_count(thr_cut)
            smem[1] = c_hi
            smem[2] = blk_count(t_lo) - c_hi
            smem[3] = t_lo
            smem[4] = t_hi

        @pl.when(i > 0)
        def _rest():
            t_lo = smem[3]
            t_hi = smem[4]
            c_hi = blk_count(t_hi)
            smem[0] = smem[0] + blk_count(thr_cut)
            smem[1] = smem[1] + c_hi
            smem[2] = smem[2] + blk_count(t_lo) - c_hi

    @pl.when(i == _G)
    def _phase_b():
        num_updates = smem[0]
        n_above = smem[1]
        n_in = smem[2]
        t_lo = smem[3]
        t_hi = smem[4]

        ok = jnp.logical_and(n_above < _K_TARGET,
                             n_above + n_in >= _K_TARGET)
        lo0 = jnp.where(ok, t_lo, jnp.int32(0))
        hi0 = jnp.where(ok, t_hi, jnp.int32(_INF_BITS))
        c0 = jnp.where(ok, n_above + n_in, jnp.int32(_N))

        bits = bits_scr[...]
        ones_row = jnp.ones((1, _NUM_PRE), jnp.float32)

        def cond(state):
            lo, hi, c_lo = state
            return jnp.logical_and(c_lo > _K_TARGET + _RANK_TOL, hi - lo > 1)

        def body(state):
            lo, hi, c_lo = state
            mid = lo + (hi - lo) // 2
            maskf = jnp.where(bits >= mid, 1.0, 0.0)
            col = jnp.dot(ones_row, maskf,
                          preferred_element_type=jnp.float32)
            c = jnp.sum(col).astype(jnp.int32)
            ge = c >= _K_TARGET
            return (jnp.where(ge, mid, lo), jnp.where(ge, hi, mid),
                    jnp.where(ge, c, c_lo))

        tstar, _, _ = lax.while_loop(cond, body, (lo0, hi0, c0))

        use_topk = num_updates > _K_TARGET
        smem[5] = jnp.where(use_topk, tstar, thr_cut)

    @pl.when(i >= _G)
    def _phase_c():
        j = i - _G
        cut = smem[5]
        ne = ne_scr[pl.ds(j * _BLK, _BLK), :]
        abits = bits_scr[pl.ds(j * _BLK, _BLK), :]
        mask = abits >= cut
        wu_ref[...] = jnp.where(mask, jnp.clip(ne, -mwc, mwc),
                                jnp.zeros_like(ne))


def kernel(pre_spikes, post_spikes, weights, eligibility_trace, a_plus,
           a_minus, tau_plus, tau_minus, tau_eligibility, activity_threshold,
           max_weight_change):
    del weights, tau_plus, tau_minus
    decay = jnp.exp(-_DT / tau_eligibility)
    scale = (a_plus + a_minus) / _BATCH
    scalars = jnp.stack([decay, scale, activity_threshold,
                         max_weight_change]).astype(jnp.float32)
    pre_t = pre_spikes.T

    out_shape = (
        jax.ShapeDtypeStruct((_NUM_PRE, _NUM_POST), jnp.float32),
        jax.ShapeDtypeStruct((_NUM_PRE, _NUM_POST), jnp.float32),
    )
    wu, new_elig = pl.pallas_call(
        _body,
        grid=(2 * _G,),
        out_shape=out_shape,
        in_specs=[
            pl.BlockSpec(memory_space=pltpu.SMEM),
            pl.BlockSpec((_BLK, _BATCH),
                         lambda i: (jnp.minimum(i, _G - 1), 0)),
            pl.BlockSpec((_BATCH, _NUM_POST), lambda i: (0, 0)),
            pl.BlockSpec((_BLK, _NUM_POST),
                         lambda i: (jnp.minimum(i, _G - 1), 0)),
        ],
        out_specs=(
            pl.BlockSpec((_BLK, _NUM_POST),
                         lambda i: (jnp.maximum(i - _G, 0), 0)),
            pl.BlockSpec((_BLK, _NUM_POST),
                         lambda i: (jnp.minimum(i, _G - 1), 0)),
        ),
        scratch_shapes=[
            pltpu.VMEM((_NUM_PRE, _NUM_POST), jnp.float32),
            pltpu.VMEM((_NUM_PRE, _NUM_POST), jnp.int32),
            pltpu.SMEM((8,), jnp.int32),
        ],
        compiler_params=pltpu.CompilerParams(
            dimension_semantics=("arbitrary",)),
    )(scalars, pre_t, post_spikes, eligibility_trace)
    return (wu, new_elig)


# speedup vs baseline: 3.3529x; 1.0300x over previous
import jax
import jax.numpy as jnp
from jax import lax
from jax.experimental import pallas as pl
from jax.experimental.pallas import tpu as pltpu

_NUM_PRE = 2048
_NUM_POST = 1024
_BATCH = 16
_N = _NUM_PRE * _NUM_POST
_K_TARGET = int(0.1 * _N)
_DT = 0.1
_BLK = 1024
_G = _NUM_PRE // _BLK

_SUB_ROWS = 64
_K_SUB = _K_TARGET // 32
_M_SUB = 480
_SUB_ITERS = 22
_RANK_TOL = 6
_INF_BITS = 0x7F800000


def _sub_bisect(data, target, n_iter):
    def step(_, lohi):
        lo, hi = lohi
        mid = lo + (hi - lo) // 2
        c = jnp.sum((data >= mid).astype(jnp.int32))
        ge = c >= target
        return jnp.where(ge, mid, lo), jnp.where(ge, hi, mid)

    lo, _ = lax.fori_loop(0, n_iter, step, (jnp.int32(0),
                                            jnp.int32(_INF_BITS)))
    return lo


def _body(scal_ref, pre_blk_ref, post_ref, elig_blk_ref, wu_ref, elig_out_ref,
          ne_scr, bits_scr, smem):
    i = pl.program_id(0)
    decay = scal_ref[0]
    scale = scal_ref[1]
    thr = scal_ref[2]
    mwc = scal_ref[3]

    thr_bits = jax.lax.bitcast_convert_type(thr, jnp.int32)
    thr_cut = jnp.where(thr >= 0.0, thr_bits + 1, jnp.int32(0))

    @pl.when(i < _G)
    def _phase_a():
        upd = jnp.dot(pre_blk_ref[...], post_ref[...],
                      preferred_element_type=jnp.float32) * scale
        ne = elig_blk_ref[...] * decay + upd
        elig_out_ref[...] = ne
        act = jnp.abs(ne)
        abits = jax.lax.bitcast_convert_type(act, jnp.int32)
        ne_scr[pl.ds(i * _BLK, _BLK), :] = ne
        bits_scr[pl.ds(i * _BLK, _BLK), :] = abits

        ones_blk = jnp.ones((1, _BLK), jnp.float32)

        def blk_count(t):
            maskf = jnp.where(abits >= t, 1.0, 0.0)
            col = jnp.dot(ones_blk, maskf,
                          preferred_element_type=jnp.float32)
            return jnp.sum(col).astype(jnp.int32)

        @pl.when(i == 0)
        def _first():
            sub = abits[0:_SUB_ROWS, :]
            t_lo = _sub_bisect(sub, _K_SUB + _M_SUB, _SUB_ITERS)
            t_hi = _sub_bisect(sub, _K_SUB - _M_SUB, _SUB_ITERS)
            c_hi = blk_count(t_hi)
            smem[0] = blk_count(thr_cut)
            smem[1] = c_hi
            smem[2] = blk_count(t_lo) - c_hi
            smem[3] = t_lo
            smem[4] = t_hi

        @pl.when(i > 0)
        def _rest():
            t_lo = smem[3]
            t_hi = smem[4]
            c_hi = blk_count(t_hi)
            smem[0] = smem[0] + blk_count(thr_cut)
            smem[1] = smem[1] + c_hi
            smem[2] = smem[2] + blk_count(t_lo) - c_hi

    @pl.when(i == _G)
    def _phase_b():
        num_updates = smem[0]
        n_above = smem[1]
        n_in = smem[2]
        t_lo = smem[3]
        t_hi = smem[4]

        ok = jnp.logical_and(n_above < _K_TARGET,
                             n_above + n_in >= _K_TARGET)
        lo0 = jnp.where(ok, t_lo, jnp.int32(0))
        hi0 = jnp.where(ok, t_hi, jnp.int32(_INF_BITS))
        c0 = jnp.where(ok, n_above + n_in, jnp.int32(_N))

        bits = bits_scr[...]
        ones_row = jnp.ones((1, _NUM_PRE), jnp.float32)

        def cond(state):
            lo, hi, c_lo = state
            return jnp.logical_and(c_lo > _K_TARGET + _RANK_TOL, hi - lo > 1)

        def body(state):
            lo, hi, c_lo = state
            mid = lo + (hi - lo) // 2
            maskf = jnp.where(bits >= mid, 1.0, 0.0)
            col = jnp.dot(ones_row, maskf,
                          preferred_element_type=jnp.float32)
            c = jnp.sum(col).astype(jnp.int32)
            ge = c >= _K_TARGET
            return (jnp.where(ge, mid, lo), jnp.where(ge, hi, mid),
                    jnp.where(ge, c, c_lo))

        tstar, _, _ = lax.while_loop(cond, body, (lo0, hi0, c0))

        use_topk = num_updates > _K_TARGET
        smem[5] = jnp.where(use_topk, tstar, thr_cut)

    @pl.when(i >= _G)
    def _phase_c():
        j = i - _G
        cut = smem[5]
        ne = ne_scr[pl.ds(j * _BLK, _BLK), :]
        abits = bits_scr[pl.ds(j * _BLK, _BLK), :]
        mask = abits >= cut
        wu_ref[...] = jnp.where(mask, jnp.clip(ne, -mwc, mwc),
                                jnp.zeros_like(ne))


def kernel(pre_spikes, post_spikes, weights, eligibility_trace, a_plus,
           a_minus, tau_plus, tau_minus, tau_eligibility, activity_threshold,
           max_weight_change):
    del weights, tau_plus, tau_minus
    decay = jnp.exp(-_DT / tau_eligibility)
    scale = (a_plus + a_minus) / _BATCH
    scalars = jnp.stack([decay, scale, activity_threshold,
                         max_weight_change]).astype(jnp.float32)
    pre_t = pre_spikes.T

    out_shape = (
        jax.ShapeDtypeStruct((_NUM_PRE, _NUM_POST), jnp.float32),
        jax.ShapeDtypeStruct((_NUM_PRE, _NUM_POST), jnp.float32),
    )
    wu, new_elig = pl.pallas_call(
        _body,
        grid=(2 * _G,),
        out_shape=out_shape,
        in_specs=[
            pl.BlockSpec(memory_space=pltpu.SMEM),
            pl.BlockSpec((_BLK, _BATCH),
                         lambda i: (jnp.minimum(i, _G - 1), 0)),
            pl.BlockSpec((_BATCH, _NUM_POST), lambda i: (0, 0)),
            pl.BlockSpec((_BLK, _NUM_POST),
                         lambda i: (jnp.minimum(i, _G - 1), 0)),
        ],
        out_specs=(
            pl.BlockSpec((_BLK, _NUM_POST),
                         lambda i: (jnp.maximum(i - _G, 0), 0)),
            pl.BlockSpec((_BLK, _NUM_POST),
                         lambda i: (jnp.minimum(i, _G - 1), 0)),
        ),
        scratch_shapes=[
            pltpu.VMEM((_NUM_PRE, _NUM_POST), jnp.float32),
            pltpu.VMEM((_NUM_PRE, _NUM_POST), jnp.int32),
            pltpu.SMEM((8,), jnp.int32),
        ],
        compiler_params=pltpu.CompilerParams(
            dimension_semantics=("arbitrary",)),
    )(scalars, pre_t, post_spikes, eligibility_trace)
    return (wu, new_elig)
